# baseline (speedup 1.0000x reference)
import functools

import jax
import jax.numpy as jnp
from jax import lax
from jax.experimental import pallas as pl
from jax.experimental.pallas import tpu as pltpu
from jax.experimental.pallas import tpu_sc as plsc

_B, _N, _E = 32, 64, 256
_FIN, _FB, _FOUT = 32, 16, 32
_L = 16
_W = 2 * _N + _E


def _tc_dense(sites_t, bonds4, W1, b1col, wattcol, batt11):
    def body(st_hbm, bn_hbm, w1_ref, b1_ref, wa_ref, ba_ref,
             xc_ref, watt_ref, batt_ref, st_v, bn_v, sem1, sem2):
        cp1 = pltpu.make_async_copy(st_hbm, st_v, sem1)
        cp2 = pltpu.make_async_copy(bn_hbm, bn_v, sem2)
        cp1.start()
        cp2.start()
        cp1.wait()
        cp2.wait()
        w1 = w1_ref[...]
        wab = jnp.concatenate([w1[:, _FIN:2 * _FIN], w1[:, 0:_FIN]], axis=0)
        wc = w1[:, 2 * _FIN:]
        b1 = b1_ref[...]
        st_all = jnp.concatenate([st_v[b] for b in range(_B)], axis=1)
        xx = lax.dot_general(wab, st_all, (((1,), (0,)), ((), ())),
                             preferred_element_type=jnp.float32)
        qall = lax.dot_general(wc, bn_v[...], (((1,), (1,)), ((), ())),
                               preferred_element_type=jnp.float32)
        for b in range(_B):
            lo, hi = b * _N, (b + 1) * _N
            xr = xx[0:_FOUT, lo:hi]
            xa = xx[_FOUT:2 * _FOUT, lo:hi] + b1
            xc_ref[b] = jnp.concatenate(
                [xr, xa, qall[:, b * _E:(b + 1) * _E]], axis=1)
        watt_ref[...] = jnp.broadcast_to(wa_ref[...], (_FOUT, _L))
        batt_ref[...] = jnp.broadcast_to(ba_ref[...], (1, _L))

    return pl.pallas_call(
        body,
        in_specs=[
            pl.BlockSpec(memory_space=pltpu.MemorySpace.HBM),
            pl.BlockSpec(memory_space=pltpu.MemorySpace.HBM),
            pl.BlockSpec(memory_space=pltpu.MemorySpace.VMEM),
            pl.BlockSpec(memory_space=pltpu.MemorySpace.VMEM),
            pl.BlockSpec(memory_space=pltpu.MemorySpace.VMEM),
            pl.BlockSpec(memory_space=pltpu.MemorySpace.VMEM),
        ],
        scratch_shapes=[
            pltpu.VMEM((_B, _FIN, _N), jnp.float32),
            pltpu.VMEM((_B * _E, _FB), jnp.float32),
            pltpu.SemaphoreType.DMA,
            pltpu.SemaphoreType.DMA,
        ],
        out_shape=[
            jax.ShapeDtypeStruct((_B, _FOUT, _W), jnp.float32),
            jax.ShapeDtypeStruct((_FOUT, _L), jnp.float32),
            jax.ShapeDtypeStruct((1, _L), jnp.float32),
        ],
    )(sites_t, bonds4, W1, b1col, wattcol, batt11)


def _sc_messages(xc, idx2d, watt, batt):
    mesh = plsc.VectorSubcoreMesh(core_axis_name="c", subcore_axis_name="s")

    @functools.partial(
        pl.kernel,
        mesh=mesh,
        compiler_params=pltpu.CompilerParams(needs_layout_passes=False),
        out_type=jax.ShapeDtypeStruct((_B, _FOUT, _N), jnp.float32),
        scratch_types=[
            pltpu.VMEM((_FOUT, _W), jnp.float32),
            pltpu.VMEM((4 * _N,), jnp.int32),
            pltpu.VMEM((_FOUT, _L), jnp.float32),
            pltpu.VMEM((1, _L), jnp.float32),
            pltpu.VMEM((_FOUT, _L), jnp.float32),
            pltpu.VMEM((_FOUT, _N), jnp.float32),
            pltpu.SemaphoreType.DMA,
            pltpu.SemaphoreType.DMA,
            pltpu.SemaphoreType.DMA,
            pltpu.SemaphoreType.DMA,
        ],
    )
    def k(xc_hbm, idx_hbm, watt_hbm, batt_hbm, out_hbm,
          xc_v, idx_v, watt_v, batt_v, lat_v, out_v,
          sem1, sem2, sem3, sem4):
        b = lax.axis_index("s") * 2 + lax.axis_index("c")
        c1 = pltpu.async_copy(xc_hbm.at[b], xc_v, sem1)
        c2 = pltpu.async_copy(idx_hbm, idx_v, sem2)
        c3 = pltpu.async_copy(watt_hbm, watt_v, sem3)
        c4 = pltpu.async_copy(batt_hbm, batt_v, sem4)
        c1.wait()
        c2.wait()
        c3.wait()
        c4.wait()

        zero = jnp.zeros((_L,), jnp.float32)
        lanes = jax.lax.iota(jnp.int32, _L)

        @plsc.parallel_loop(0, _FOUT, unroll=4)
        def _(o):
            oc = jnp.full((_L,), o, jnp.int32)
            for nk in range(_N // _L):
                plsc.store_scatter(out_v, [oc, nk * _L + lanes], zero)

        def chunk_body(dk, _c):
            base = dk * _L
            kk = dk & 3
            d = dk >> 2
            scol = _N + kk * _L + lanes
            eidx = 2 * _N + (kk * _L + lanes) * 4 + d
            v2 = idx_v[pl.ds(base, _L)]

            def msg(o, accs):
                oc = jnp.full((_L,), o, jnp.int32)
                g1 = plsc.load_gather(xc_v, [oc, scol])
                q = plsc.load_gather(xc_v, [oc, eidx])
                g2 = plsc.load_gather(xc_v, [oc, v2])
                pre = (g1 + q) + g2
                lt = jnp.where(pre >= 0.0, pre, 0.01 * pre)
                plsc.store_scatter(lat_v, [oc, lanes], lt)
                w = plsc.load_gather(watt_v, [oc, lanes])
                a0, a1, a2, a3 = accs
                return (a1, a2, a3, a0 + lt * w)

            accs = plsc.parallel_loop(
                0, _FOUT, unroll=8,
                carry=(batt_v[0, :], zero, zero, zero))(msg)
            acc = (accs[0] + accs[1]) + (accs[2] + accs[3])
            att = 1.0 / (1.0 + jnp.exp(-acc))

            @plsc.parallel_loop(0, _FOUT, unroll=4)
            def _(o):
                oc = jnp.full((_L,), o, jnp.int32)
                lt = plsc.load_gather(lat_v, [oc, lanes])
                plsc.addupdate_scatter(out_v, [oc, v2], lt * att)

            return 0

        lax.fori_loop(0, 16, chunk_body, 0)

        pltpu.sync_copy(out_v, out_hbm.at[b])

    return k(xc, idx2d, watt, batt)


def kernel(sites, bonds, W1, b1, w_att, b_att, idx1, idx2, idx2_oh):
    b1col = b1.reshape(_FOUT, 1)
    wattcol = w_att.reshape(_FOUT, 1)
    batt11 = b_att.reshape(1, 1).astype(jnp.float32)
    sites_t = jnp.swapaxes(sites, 1, 2)
    bonds2d = bonds.reshape(_B * _E, _FB)
    idx2d = idx2.reshape(_N, 4).T.reshape(-1)

    xc, watt, batt = _tc_dense(sites_t, bonds2d, W1, b1col, wattcol, batt11)
    outT = _sc_messages(xc, idx2d, watt, batt)
    return jnp.swapaxes(outT, 1, 2)


# speedup vs baseline: 1.0024x; 1.0024x over previous
import functools

import jax
import jax.numpy as jnp
from jax import lax
from jax.experimental import pallas as pl
from jax.experimental.pallas import tpu as pltpu
from jax.experimental.pallas import tpu_sc as plsc

_B, _N, _E = 32, 64, 256
_FIN, _FB, _FOUT = 32, 16, 32
_L = 16
_W = 2 * _N + _E


def _tc_dense(sites_t, bonds4, W1, b1col, wattcol, batt11):
    def body(st_hbm, bn_hbm, w1_ref, b1_ref, wa_ref, ba_ref,
             xc_ref, watt_ref, batt_ref, st_v, bn_v, sem1, sem2):
        cp1 = pltpu.make_async_copy(st_hbm, st_v, sem1)
        cp2 = pltpu.make_async_copy(bn_hbm, bn_v, sem2)
        cp1.start()
        cp2.start()
        cp1.wait()
        cp2.wait()
        w1 = w1_ref[...]
        wab = jnp.concatenate([w1[:, _FIN:2 * _FIN], w1[:, 0:_FIN]], axis=0)
        wc = w1[:, 2 * _FIN:]
        b1 = b1_ref[...]
        st_all = jnp.concatenate([st_v[b] for b in range(_B)], axis=1)
        xx = lax.dot_general(wab, st_all, (((1,), (0,)), ((), ())),
                             preferred_element_type=jnp.float32)
        qall = lax.dot_general(wc, bn_v[...], (((1,), (1,)), ((), ())),
                               preferred_element_type=jnp.float32)
        for b in range(_B):
            lo, hi = b * _N, (b + 1) * _N
            xr = xx[0:_FOUT, lo:hi]
            xa = xx[_FOUT:2 * _FOUT, lo:hi] + b1
            xc_ref[b] = jnp.concatenate(
                [xr, xa, qall[:, b * _E:(b + 1) * _E]], axis=1)
        watt_ref[...] = jnp.broadcast_to(wa_ref[...], (_FOUT, _L))
        batt_ref[...] = jnp.broadcast_to(ba_ref[...], (1, _L))

    return pl.pallas_call(
        body,
        in_specs=[
            pl.BlockSpec(memory_space=pltpu.MemorySpace.HBM),
            pl.BlockSpec(memory_space=pltpu.MemorySpace.HBM),
            pl.BlockSpec(memory_space=pltpu.MemorySpace.VMEM),
            pl.BlockSpec(memory_space=pltpu.MemorySpace.VMEM),
            pl.BlockSpec(memory_space=pltpu.MemorySpace.VMEM),
            pl.BlockSpec(memory_space=pltpu.MemorySpace.VMEM),
        ],
        scratch_shapes=[
            pltpu.VMEM((_B, _FIN, _N), jnp.float32),
            pltpu.VMEM((_B * _E, _FB), jnp.float32),
            pltpu.SemaphoreType.DMA,
            pltpu.SemaphoreType.DMA,
        ],
        out_shape=[
            jax.ShapeDtypeStruct((_B, _FOUT, _W), jnp.float32),
            jax.ShapeDtypeStruct((_FOUT, _L), jnp.float32),
            jax.ShapeDtypeStruct((1, _L), jnp.float32),
        ],
    )(sites_t, bonds4, W1, b1col, wattcol, batt11)


def _sc_messages(xc, idx2d, watt, batt):
    mesh = plsc.VectorSubcoreMesh(core_axis_name="c", subcore_axis_name="s")

    @functools.partial(
        pl.kernel,
        mesh=mesh,
        compiler_params=pltpu.CompilerParams(needs_layout_passes=False),
        out_type=jax.ShapeDtypeStruct((_B, _FOUT, _N), jnp.float32),
        scratch_types=[
            pltpu.VMEM((_FOUT, _W), jnp.float32),
            pltpu.VMEM((4 * _N,), jnp.int32),
            pltpu.VMEM((_FOUT, _L), jnp.float32),
            pltpu.VMEM((1, _L), jnp.float32),
            pltpu.VMEM((_FOUT, _L), jnp.float32),
            pltpu.VMEM((_FOUT, _N), jnp.float32),
            pltpu.SemaphoreType.DMA,
            pltpu.SemaphoreType.DMA,
            pltpu.SemaphoreType.DMA,
            pltpu.SemaphoreType.DMA,
        ],
    )
    def k(xc_hbm, idx_hbm, watt_hbm, batt_hbm, out_hbm,
          xc_v, idx_v, watt_v, batt_v, lat_v, out_v,
          sem1, sem2, sem3, sem4):
        b = lax.axis_index("s") * 2 + lax.axis_index("c")
        c1 = pltpu.async_copy(xc_hbm.at[b], xc_v, sem1)
        c2 = pltpu.async_copy(idx_hbm, idx_v, sem2)
        c3 = pltpu.async_copy(watt_hbm, watt_v, sem3)
        c4 = pltpu.async_copy(batt_hbm, batt_v, sem4)
        c1.wait()
        c2.wait()
        c3.wait()
        c4.wait()

        zero = jnp.zeros((_L,), jnp.float32)
        lanes = jax.lax.iota(jnp.int32, _L)

        @plsc.parallel_loop(0, _FOUT, unroll=4)
        def _(o):
            oc = jnp.full((_L,), o, jnp.int32)
            for nk in range(_N // _L):
                plsc.store_scatter(out_v, [oc, nk * _L + lanes], zero)

        def chunk_body(dk, _c):
            base = dk * _L
            kk = dk & 3
            d = dk >> 2
            scol = _N + kk * _L + lanes
            eidx = 2 * _N + (kk * _L + lanes) * 4 + d
            v2 = idx_v[pl.ds(base, _L)]

            def msg(o, accs):
                oc = jnp.full((_L,), o, jnp.int32)
                g1 = plsc.load_gather(xc_v, [oc, scol])
                q = plsc.load_gather(xc_v, [oc, eidx])
                g2 = plsc.load_gather(xc_v, [oc, v2])
                pre = (g1 + q) + g2
                lt = jnp.where(pre >= 0.0, pre, 0.01 * pre)
                plsc.store_scatter(lat_v, [oc, lanes], lt)
                w = plsc.load_gather(watt_v, [oc, lanes])
                a0, a1, a2, a3 = accs
                return (a1, a2, a3, a0 + lt * w)

            accs = plsc.parallel_loop(
                0, _FOUT, unroll=4,
                carry=(batt_v[0, :], zero, zero, zero))(msg)
            acc = (accs[0] + accs[1]) + (accs[2] + accs[3])
            att = 1.0 / (1.0 + jnp.exp(-acc))

            @plsc.parallel_loop(0, _FOUT, unroll=4)
            def _(o):
                oc = jnp.full((_L,), o, jnp.int32)
                lt = plsc.load_gather(lat_v, [oc, lanes])
                plsc.addupdate_scatter(out_v, [oc, v2], lt * att)

            return 0

        lax.fori_loop(0, 16, chunk_body, 0)

        pltpu.sync_copy(out_v, out_hbm.at[b])

    return k(xc, idx2d, watt, batt)


def kernel(sites, bonds, W1, b1, w_att, b_att, idx1, idx2, idx2_oh):
    b1col = b1.reshape(_FOUT, 1)
    wattcol = w_att.reshape(_FOUT, 1)
    batt11 = b_att.reshape(1, 1).astype(jnp.float32)
    sites_t = jnp.swapaxes(sites, 1, 2)
    bonds2d = bonds.reshape(_B * _E, _FB)
    idx2d = idx2.reshape(_N, 4).T.reshape(-1)

    xc, watt, batt = _tc_dense(sites_t, bonds2d, W1, b1col, wattcol, batt11)
    outT = _sc_messages(xc, idx2d, watt, batt)
    return jnp.swapaxes(outT, 1, 2)
